# baseline (speedup 1.0000x reference)
import functools

import jax
import jax.numpy as jnp
from jax import lax
from jax.experimental import pallas as pl
from jax.experimental.pallas import tpu as pltpu
from jax.experimental.pallas import tpu_sc as plsc

_NW = 32
_OUTER = 1600
_GW = 80
_NG = _OUTER // _GW
_STEPS = _OUTER // 16


def _sc_body(xflat_hbm, table_hbm, out_hbm, xbuf, idxbuf, valbuf, sem,
             *, nchunk):
    wid = lax.axis_index("s") * 2 + lax.axis_index("c")
    lanes3 = lax.iota(jnp.int32, 16) * 3
    nloc = (nchunk - 1 - wid) // _NW + 1

    def body(i, carry):
        c = wid + i * _NW
        xoff = pl.multiple_of(c * (_OUTER * 3), 8)
        ooff = pl.multiple_of(c * _OUTER, 8)
        pltpu.sync_copy(xflat_hbm.at[pl.ds(xoff, _OUTER * 3)], xbuf)
        for s in range(_STEPS):
            base = lanes3 + (48 * s)
            x0 = plsc.load_gather(xbuf, [base])
            x1 = plsc.load_gather(xbuf, [base + 1])
            x2 = plsc.load_gather(xbuf, [base + 2])
            flat = (x0 << 16) | (x1 << 8) | x2
            idxbuf[pl.ds(16 * s, 16)] = flat
        cps = [
            pltpu.async_copy(
                table_hbm.at[idxbuf.at[pl.ds(j * _GW, _GW)]],
                valbuf.at[pl.ds(j * _GW, _GW)],
                sem,
            )
            for j in range(_NG)
        ]
        for cp in cps:
            cp.wait()
        pltpu.sync_copy(valbuf, out_hbm.at[pl.ds(ooff, _OUTER)])
        return carry

    lax.fori_loop(0, nloc, body, 0)


def kernel(X, probs):
    n = X.shape[0]
    xflat = X.astype(jnp.int32).reshape(-1)
    table = probs.reshape(-1)
    nchunk = n // _OUTER
    mesh = plsc.VectorSubcoreMesh(core_axis_name="c", subcore_axis_name="s")
    run = functools.partial(
        pl.kernel,
        mesh=mesh,
        out_type=jax.ShapeDtypeStruct((n,), jnp.float32),
        scratch_types=[
            pltpu.VMEM((_OUTER * 3,), jnp.int32),
            pltpu.VMEM((_OUTER,), jnp.int32),
            pltpu.VMEM((_OUTER,), jnp.float32),
            pltpu.SemaphoreType.DMA,
        ],
        compiler_params=pltpu.CompilerParams(needs_layout_passes=False),
    )(functools.partial(_sc_body, nchunk=nchunk))
    return run(xflat, table)


# speedup vs baseline: 22.9151x; 22.9151x over previous
import functools

import jax
import jax.numpy as jnp
from jax import lax
from jax.experimental import pallas as pl
from jax.experimental.pallas import tpu as pltpu
from jax.experimental.pallas import tpu_sc as plsc

_NW = 32
_OUTER = 1600
_GW = 80
_NG = _OUTER // _GW
_STEPS = _OUTER // 16


def _sc_body(xt_hbm, table_hbm, out_hbm, x0b, x1b, x2b, idxbuf, valbuf, sem,
             *, nchunk):
    wid = lax.axis_index("s") * 2 + lax.axis_index("c")
    nloc = (nchunk - 1 - wid) // _NW + 1

    def body(i, carry):
        c = wid + i * _NW
        ooff = pl.multiple_of(c * _OUTER, 8)
        pltpu.sync_copy(xt_hbm.at[0, pl.ds(ooff, _OUTER)], x0b)
        pltpu.sync_copy(xt_hbm.at[1, pl.ds(ooff, _OUTER)], x1b)
        pltpu.sync_copy(xt_hbm.at[2, pl.ds(ooff, _OUTER)], x2b)
        for s in range(_STEPS):
            sl = pl.ds(16 * s, 16)
            a = x0b[sl]
            b = x1b[sl]
            k = x2b[sl]
            off = (
                (a << 16)
                | ((b & ~7) << 8)
                | ((b & 7) << 7)
                | ((k & 128) << 3)
                | (k & 127)
            )
            idxbuf[sl] = off
        cps = [
            pltpu.async_copy(
                table_hbm.at[idxbuf.at[pl.ds(j * _GW, _GW)]],
                valbuf.at[pl.ds(j * _GW, _GW)],
                sem,
            )
            for j in range(_NG)
        ]
        for cp in cps:
            cp.wait()
        pltpu.sync_copy(valbuf, out_hbm.at[pl.ds(ooff, _OUTER)])
        return carry

    lax.fori_loop(0, nloc, body, 0)


def kernel(X, probs):
    n = X.shape[0]
    xt = X.astype(jnp.int32).T
    table = (
        probs.reshape(256, 32, 8, 2, 128)
        .transpose(0, 1, 3, 2, 4)
        .reshape(-1)
    )
    nchunk = n // _OUTER
    mesh = plsc.VectorSubcoreMesh(core_axis_name="c", subcore_axis_name="s")
    run = functools.partial(
        pl.kernel,
        mesh=mesh,
        out_type=jax.ShapeDtypeStruct((n,), jnp.float32),
        scratch_types=[
            pltpu.VMEM((_OUTER,), jnp.int32),
            pltpu.VMEM((_OUTER,), jnp.int32),
            pltpu.VMEM((_OUTER,), jnp.int32),
            pltpu.VMEM((_OUTER,), jnp.int32),
            pltpu.VMEM((_OUTER,), jnp.float32),
            pltpu.SemaphoreType.DMA,
        ],
        compiler_params=pltpu.CompilerParams(
            needs_layout_passes=False, use_tc_tiling_on_sc=False
        ),
    )(functools.partial(_sc_body, nchunk=nchunk))
    return run(xt, table)


# speedup vs baseline: 34.3402x; 1.4986x over previous
import functools

import jax
import jax.numpy as jnp
from jax import lax
from jax.experimental import pallas as pl
from jax.experimental.pallas import tpu as pltpu
from jax.experimental.pallas import tpu_sc as plsc

_NW = 32
_OUTER = 8000
_STEPS = _OUTER // 16
_UNROLL = 4
_SLOTS = 4
_GWS = [128] * (_OUTER // 128) + ([_OUTER % 128] if _OUTER % 128 else [])


def _sc_body(xt_hbm, table_hbm, out_hbm,
             xb0, xb1, idx0, idx1, val0, val1,
             semx0, semx1, semg0, semg1, semo0, semo1,
             *, nchunk):
    wid = lax.axis_index("s") * 2 + lax.axis_index("c")
    xb = [xb0, xb1]
    idxb = [idx0, idx1]
    valb = [val0, val1]
    semx = [semx0, semx1]
    semg = [semg0, semg1]
    semo = [semo0, semo1]

    def off_of(g):
        cc = wid + g * _NW
        c = jnp.where(cc < nchunk, cc, cc - nchunk)
        return pl.multiple_of(c * _OUTER, 8)

    def x_copies(g):
        ph = g % 2
        off = off_of(g)
        return [
            pltpu.make_async_copy(xt_hbm.at[c, pl.ds(off, _OUTER)],
                                  xb[ph].at[c], semx[ph])
            for c in range(3)
        ]

    def gather_copies(g):
        ph = g % 2
        cps = []
        base = 0
        for gw in _GWS:
            cps.append(pltpu.make_async_copy(
                table_hbm.at[idxb[ph].at[pl.ds(base, gw)]],
                valb[ph].at[pl.ds(base, gw)],
                semg[ph],
            ))
            base += gw
        return cps

    def out_copy(g):
        ph = g % 2
        return pltpu.make_async_copy(valb[ph],
                                     out_hbm.at[pl.ds(off_of(g), _OUTER)],
                                     semo[ph])

    def compute(g):
        ph = g % 2
        xr, ir = xb[ph], idxb[ph]

        def step(s, carry):
            for u in range(_UNROLL):
                sl = pl.ds(pl.multiple_of(16 * (_UNROLL * s + u), 16), 16)
                a = xr[0, sl]
                b = xr[1, sl]
                k = xr[2, sl]
                ir[sl] = (
                    (a << 16)
                    | ((b & ~7) << 8)
                    | ((b & 7) << 7)
                    | ((k & 128) << 3)
                    | (k & 127)
                )
            return carry

        lax.fori_loop(0, _STEPS // _UNROLL, step, 0)

    for cp in x_copies(0):
        cp.start()
    for g in range(_SLOTS):
        if g + 1 < _SLOTS:
            for cp in x_copies(g + 1):
                cp.start()
        for cp in x_copies(g):
            cp.wait()
        compute(g)
        if g >= 1:
            for cp in gather_copies(g - 1):
                cp.wait()
            out_copy(g - 1).start()
        if g >= 2:
            out_copy(g - 2).wait()
        for cp in gather_copies(g):
            cp.start()
    g_last = _SLOTS - 1
    for cp in gather_copies(g_last):
        cp.wait()
    out_copy(g_last).start()
    out_copy(g_last - 1).wait()
    out_copy(g_last).wait()


def kernel(X, probs):
    n = X.shape[0]
    xt = X.astype(jnp.int32).T
    table = (
        probs.reshape(256, 32, 8, 2, 128)
        .transpose(0, 1, 3, 2, 4)
        .reshape(-1)
    )
    nchunk = n // _OUTER
    mesh = plsc.VectorSubcoreMesh(core_axis_name="c", subcore_axis_name="s")
    run = functools.partial(
        pl.kernel,
        mesh=mesh,
        out_type=jax.ShapeDtypeStruct((n,), jnp.float32),
        scratch_types=[
            pltpu.VMEM((3, _OUTER), jnp.int32),
            pltpu.VMEM((3, _OUTER), jnp.int32),
            pltpu.VMEM((_OUTER,), jnp.int32),
            pltpu.VMEM((_OUTER,), jnp.int32),
            pltpu.VMEM((_OUTER,), jnp.float32),
            pltpu.VMEM((_OUTER,), jnp.float32),
            pltpu.SemaphoreType.DMA,
            pltpu.SemaphoreType.DMA,
            pltpu.SemaphoreType.DMA,
            pltpu.SemaphoreType.DMA,
            pltpu.SemaphoreType.DMA,
            pltpu.SemaphoreType.DMA,
        ],
        compiler_params=pltpu.CompilerParams(
            needs_layout_passes=False, use_tc_tiling_on_sc=False
        ),
    )(functools.partial(_sc_body, nchunk=nchunk))
    return run(xt, table)
